# baseline (speedup 1.0000x reference)
import functools

import jax
import jax.numpy as jnp
from jax import lax
from jax.experimental import pallas as pl
from jax.experimental.pallas import tpu as pltpu
from jax.experimental.pallas import tpu_sc as plsc

B = 16384
D = 64
NC = 2
NS = 16
NW = NC * NS
BPW = B // NW
L = 16


def _sc_scores(user_hbm, pos_hbm, neg_hbm, eu_hbm, ei_hbm, out_hbm,
               idx_u, idx_p, idx_n, u_v, p_v, n_v, acc_v, s_v,
               sem_u, sem_p, sem_n):
    wid = lax.axis_index("s") * NC + lax.axis_index("c")
    base = wid * BPW

    pltpu.sync_copy(user_hbm.at[pl.ds(base, BPW)], idx_u)
    pltpu.sync_copy(pos_hbm.at[pl.ds(base, BPW)], idx_p)
    pltpu.sync_copy(neg_hbm.at[pl.ds(base, BPW)], idx_n)

    cu = pltpu.async_copy(eu_hbm.at[idx_u], u_v, sem_u)
    cp = pltpu.async_copy(ei_hbm.at[idx_p], p_v, sem_p)
    cn = pltpu.async_copy(ei_hbm.at[idx_n], n_v, sem_n)
    cu.wait()
    cp.wait()
    cn.wait()

    def body_a(b, carry):
        a = u_v[b, pl.ds(0, L)] * (p_v[b, pl.ds(0, L)] - n_v[b, pl.ds(0, L)])
        for c in range(1, D // L):
            a = a + u_v[b, pl.ds(c * L, L)] * (
                p_v[b, pl.ds(c * L, L)] - n_v[b, pl.ds(c * L, L)])
        acc_v[b, :] = a
        return carry

    lax.fori_loop(0, BPW, body_a, 0, unroll=2)

    def body_b(g, carry):
        rows = g * L + lax.iota(jnp.int32, L)
        tot = plsc.load_gather(acc_v, [rows, jnp.zeros((L,), jnp.int32)])
        for l in range(1, L):
            tot = tot + plsc.load_gather(
                acc_v, [rows, jnp.full((L,), l, jnp.int32)])
        s_v[pl.ds(g * L, L)] = tot
        return carry

    lax.fori_loop(0, BPW // L, body_b, 0, unroll=2)

    pltpu.sync_copy(s_v, out_hbm.at[pl.ds(base, BPW)])


_sc_scores_call = functools.partial(
    pl.kernel,
    out_type=jax.ShapeDtypeStruct((B,), jnp.float32),
    mesh=plsc.VectorSubcoreMesh(core_axis_name="c", subcore_axis_name="s",
                                num_cores=NC, num_subcores=NS),
    scratch_types=[
        pltpu.VMEM((BPW,), jnp.int32),
        pltpu.VMEM((BPW,), jnp.int32),
        pltpu.VMEM((BPW,), jnp.int32),
        pltpu.VMEM((BPW, D), jnp.float32),
        pltpu.VMEM((BPW, D), jnp.float32),
        pltpu.VMEM((BPW, D), jnp.float32),
        pltpu.VMEM((BPW, L), jnp.float32),
        pltpu.VMEM((BPW,), jnp.float32),
        pltpu.SemaphoreType.DMA,
        pltpu.SemaphoreType.DMA,
        pltpu.SemaphoreType.DMA,
    ],
    compiler_params=pltpu.CompilerParams(needs_layout_passes=False,
                                         use_tc_tiling_on_sc=False),
    name="bpr_sc_scores",
)(_sc_scores)


def _tc_loss_body(s_ref, o_ref):
    s = s_ref[...]
    x = -s
    m = jnp.maximum(x, 0.0)
    sp = m + jnp.log(1.0 + jnp.exp(-jnp.abs(x)))
    o_ref[0, 0] = jnp.sum(sp) * (1.0 / B)


_tc_loss_call = pl.pallas_call(
    _tc_loss_body,
    out_shape=jax.ShapeDtypeStruct((1, 1), jnp.float32),
    in_specs=[pl.BlockSpec(memory_space=pltpu.VMEM)],
    out_specs=pl.BlockSpec(memory_space=pltpu.SMEM),
)


@jax.jit
def kernel(user, pos, neg, labels, embedding_user, embedding_item):
    del labels
    s = _sc_scores_call(user.astype(jnp.int32), pos.astype(jnp.int32),
                        neg.astype(jnp.int32), embedding_user, embedding_item)
    loss = _tc_loss_call(s.reshape(B // 128, 128))
    return loss[0, 0]
